# Initial kernel scaffold; baseline (speedup 1.0000x reference)
import jax
import jax.numpy as jnp
from jax.experimental import pallas as pl


def kernel(x1, x2, edge_index1, edge_index2, graph_id1, graph_id2, Wc1, bc1, Wc2, bc2, W1, B1, W2, B2, W3, B3, W4, B4):
    raise NotImplementedError("write your pallas kernel here")


# speedup vs baseline: 3.2218x; 3.2218x over previous
import functools

import jax
import jax.numpy as jnp
from jax import lax
from jax.experimental import pallas as pl
from jax.experimental.pallas import tpu as pltpu
from jax.experimental.pallas import tpu_sc as plsc

N = 10000
E = 320000
D = 128
G = 256
C = 2

NC = 2
NS = 16
CH = 128
KE = 160
SLAB = 32
NSLAB = KE // SLAB
EPAD = NS * KE * CH
N8 = 10240
NPW = N8 // NS
KG = 5
G8 = 512
GPW = G8 // NS
F32 = jnp.float32
I32 = jnp.int32

_BR = 1280


def _sc_degrees_body(src_hbm, dst_hbm, gid_hbm, ones_hbm, zeros_hbm,
                     dego_hbm, degi_hbm, gcnt_hbm,
                     src_v, dst_v, gid_v, ones_v, zeros_v,
                     dego_s, degi_s, gcnt_s):
    c = lax.axis_index("c")
    s = lax.axis_index("s")
    pltpu.sync_copy(src_hbm.at[c, s], src_v)
    pltpu.sync_copy(dst_hbm.at[c, s], dst_v)
    pltpu.sync_copy(gid_hbm.at[c, s], gid_v)
    pltpu.sync_copy(ones_hbm, ones_v)
    pltpu.sync_copy(zeros_hbm, zeros_v)
    base = s * NPW
    sl = pl.ds(base, NPW)
    pltpu.sync_copy(zeros_v, dego_s.at[sl])
    pltpu.sync_copy(zeros_v, degi_s.at[sl])
    pltpu.sync_copy(zeros_v.at[pl.ds(0, GPW)], gcnt_s.at[pl.ds(s * GPW, GPW)])
    plsc.subcore_barrier()

    def edge_body(j, carry):
        pltpu.sync_copy(ones_v, dego_s.at[src_v.at[j]], add=True)
        pltpu.sync_copy(ones_v, degi_s.at[dst_v.at[j]], add=True)
        return carry

    lax.fori_loop(0, KE, edge_body, 0)

    def gid_body(j, carry):
        pltpu.sync_copy(ones_v, gcnt_s.at[gid_v.at[j]], add=True)
        return carry

    lax.fori_loop(0, KG, gid_body, 0)
    plsc.subcore_barrier()
    pltpu.sync_copy(dego_s.at[sl], zeros_v)
    pltpu.sync_copy(zeros_v, dego_hbm.at[c, sl])
    pltpu.sync_copy(degi_s.at[sl], zeros_v)
    pltpu.sync_copy(zeros_v, degi_hbm.at[c, sl])

    @pl.when(s < G8 // CH)
    def _():
        gsl_in = pl.ds(s * CH, CH)
        pltpu.sync_copy(gcnt_s.at[gsl_in], zeros_v.at[pl.ds(0, CH)])
        pltpu.sync_copy(zeros_v.at[pl.ds(0, CH)], gcnt_hbm.at[c, gsl_in])


def _sc_scatter_body(table_hbm, src_hbm, dst_hbm, zeros_hbm, out_hbm,
                     src_v, dst_v, gbuf, acc_s, sem):
    c = lax.axis_index("c")
    s = lax.axis_index("s")
    pltpu.sync_copy(zeros_hbm, gbuf)
    base = s * NPW
    for k in range(NPW // CH):
        pltpu.sync_copy(gbuf, acc_s.at[pl.ds(base + k * CH, CH)])
    plsc.subcore_barrier()

    def body(j, carry):
        pltpu.async_copy(table_hbm.at[src_v.at[j]], gbuf, sem).wait()
        pltpu.sync_copy(gbuf, acc_s.at[dst_v.at[j]], add=True)
        return carry

    for t in range(NSLAB):
        pltpu.sync_copy(src_hbm.at[c, s, pl.ds(t * SLAB, SLAB)], src_v)
        pltpu.sync_copy(dst_hbm.at[c, s, pl.ds(t * SLAB, SLAB)], dst_v)
        lax.fori_loop(0, SLAB, body, 0)
    plsc.subcore_barrier()
    for k in range(NPW // CH):
        sl = pl.ds(base + k * CH, CH)
        pltpu.sync_copy(acc_s.at[sl], gbuf)
        pltpu.sync_copy(gbuf, out_hbm.at[c, sl])


@functools.lru_cache(maxsize=None)
def _sc_kernels():
    mesh = plsc.VectorSubcoreMesh(core_axis_name="c", subcore_axis_name="s")
    degrees = pl.kernel(
        _sc_degrees_body,
        out_type=(jax.ShapeDtypeStruct((NC, N8), F32),
                  jax.ShapeDtypeStruct((NC, N8), F32),
                  jax.ShapeDtypeStruct((NC, G8), F32)),
        mesh=mesh,
        scratch_types=[
            pltpu.VMEM((KE, CH), I32),
            pltpu.VMEM((KE, CH), I32),
            pltpu.VMEM((KG, CH), I32),
            pltpu.VMEM((CH,), F32),
            pltpu.VMEM((NPW,), F32),
            pltpu.VMEM_SHARED((N8,), F32),
            pltpu.VMEM_SHARED((N8,), F32),
            pltpu.VMEM_SHARED((G8,), F32),
        ],
    )
    scatter = pl.kernel(
        _sc_scatter_body,
        out_type=jax.ShapeDtypeStruct((NC, N8, D), F32),
        mesh=mesh,
        scratch_types=[
            pltpu.VMEM((SLAB, CH), I32),
            pltpu.VMEM((SLAB, CH), I32),
            pltpu.VMEM((CH, D), F32),
            pltpu.VMEM_SHARED((N8, D), F32),
            pltpu.SemaphoreType.DMA,
        ],
    )
    return degrees, scatter


def _col(v):
    return jnp.reshape(v, (v.shape[-1], 1))


def _tc_norm_body(x_ref, dego_ref, xs_ref):
    rs = _col(lax.rsqrt(jnp.maximum(dego_ref[...], 1.0)))
    xs_ref[0] = x_ref[0] * rs


_tc_norm = pl.pallas_call(
    _tc_norm_body,
    grid=(NC, N8 // _BR),
    in_specs=[pl.BlockSpec((1, _BR, D), lambda c, r: (c, r, 0)),
              pl.BlockSpec((1, 1, _BR), lambda c, r: (c, 0, r))],
    out_specs=pl.BlockSpec((1, _BR, D), lambda c, r: (c, r, 0)),
    out_shape=jax.ShapeDtypeStruct((NC, N8, D), F32),
)


def _tc_mid_body(m_ref, degi_ref, dego_ref, w_ref, b_ref, out_ref):
    rsdi = _col(lax.rsqrt(jnp.maximum(degi_ref[...], 1.0)))
    rsdo = _col(lax.rsqrt(jnp.maximum(dego_ref[...], 1.0)))
    h = jnp.dot(m_ref[0] * rsdi, w_ref[...], preferred_element_type=F32)
    h = jnp.maximum(h + b_ref[...], 0.0)
    out_ref[0] = h * rsdo


_tc_mid = pl.pallas_call(
    _tc_mid_body,
    grid=(NC, N8 // _BR),
    in_specs=[pl.BlockSpec((1, _BR, D), lambda c, r: (c, r, 0)),
              pl.BlockSpec((1, 1, _BR), lambda c, r: (c, 0, r)),
              pl.BlockSpec((1, 1, _BR), lambda c, r: (c, 0, r)),
              pl.BlockSpec((D, D), lambda c, r: (0, 0)),
              pl.BlockSpec((1, D), lambda c, r: (0, 0))],
    out_specs=pl.BlockSpec((1, _BR, D), lambda c, r: (c, r, 0)),
    out_shape=jax.ShapeDtypeStruct((NC, N8, D), F32),
)


def _tc_pool_body(m_ref, degi_ref, w_ref, b_ref, gid_ref, out_ref):
    r = pl.program_id(1)
    rsdi = _col(lax.rsqrt(jnp.maximum(degi_ref[...], 1.0)))
    h = jnp.dot(m_ref[0] * rsdi, w_ref[...], preferred_element_type=F32)
    h = jnp.maximum(h + b_ref[...], 0.0)
    gid = jnp.reshape(gid_ref[...], (1, _BR))
    oh = (lax.broadcasted_iota(I32, (G, _BR), 0) == gid).astype(F32)
    contrib = jnp.dot(oh, h, preferred_element_type=F32)

    @pl.when(r == 0)
    def _():
        out_ref[0] = contrib

    @pl.when(r != 0)
    def _():
        out_ref[0] += contrib


_tc_pool = pl.pallas_call(
    _tc_pool_body,
    grid=(NC, N8 // _BR),
    in_specs=[pl.BlockSpec((1, _BR, D), lambda c, r: (c, r, 0)),
              pl.BlockSpec((1, 1, _BR), lambda c, r: (c, 0, r)),
              pl.BlockSpec((D, D), lambda c, r: (0, 0)),
              pl.BlockSpec((1, D), lambda c, r: (0, 0)),
              pl.BlockSpec((1, 1, _BR), lambda c, r: (c, 0, r))],
    out_specs=pl.BlockSpec((1, G, D), lambda c, r: (c, 0, 0)),
    out_shape=jax.ShapeDtypeStruct((NC, G, D), F32),
    compiler_params=pltpu.CompilerParams(
        dimension_semantics=("arbitrary", "arbitrary")),
)


def _tc_mlp_body(p_ref, gcnt_ref, w1, b1, w2, b2, w3, b3, w4, b4, out_ref):
    cnt = gcnt_ref[...]
    c1 = _col(jnp.maximum(cnt[0, :, :G], 1.0))
    c2 = _col(jnp.maximum(cnt[1, :, :G], 1.0))
    hg = jnp.concatenate([p_ref[0] / c1, p_ref[1] / c2], axis=1)
    o = jnp.maximum(jnp.dot(hg, w1[...], preferred_element_type=F32) + b1[...], 0.0)
    o = jnp.maximum(jnp.dot(o, w2[...], preferred_element_type=F32) + b2[...], 0.0)
    o = jnp.maximum(jnp.dot(o, w3[...], preferred_element_type=F32) + b3[...], 0.0)
    out_ref[...] = jnp.dot(o, w4[...], preferred_element_type=F32) + b4[...]


_tc_mlp = pl.pallas_call(
    _tc_mlp_body,
    out_shape=jax.ShapeDtypeStruct((G, D), F32),
)


def kernel(x1, x2, edge_index1, edge_index2, graph_id1, graph_id2,
           Wc1, bc1, Wc2, bc2, W1, B1, W2, B2, W3, B3, W4, B4):
    def prep(ei):
        srcp = jnp.concatenate([ei[0], jnp.full((EPAD - E,), N, I32)])
        dstp = jnp.concatenate([ei[1], jnp.full((EPAD - E,), N, I32)])
        return srcp.reshape(NS, KE, CH), dstp.reshape(NS, KE, CH)

    s1, d1 = prep(edge_index1)
    s2, d2 = prep(edge_index2)
    src_l = jnp.stack([s1, s2])
    dst_l = jnp.stack([d1, d2])
    src_g = src_l + (jnp.arange(NC, dtype=I32) * N8)[:, None, None, None]
    gidp = jnp.stack([
        jnp.concatenate([graph_id1, jnp.full((N8 - N,), G, I32)]),
        jnp.concatenate([graph_id2, jnp.full((N8 - N,), G, I32)])])
    gid_sc = gidp.reshape(NC, NS, KG, CH)
    gid_tc = gidp
    x_all = jnp.zeros((NC, N8, D), F32).at[:, :N, :].set(jnp.stack([x1, x2]))
    ones1 = jnp.ones((CH,), F32)
    zeros1 = jnp.zeros((NPW,), F32)
    zeros128 = jnp.zeros((CH, D), F32)
    W4p = jnp.zeros((D, D), F32).at[:, :C].set(W4)
    B4p = jnp.zeros((1, D), F32).at[0, :C].set(B4)

    def b_(v):
        return v.reshape(1, -1)

    _sc_degrees, _sc_scatter = _sc_kernels()
    dego, degi, gcnt = _sc_degrees(src_l, dst_l, gid_sc, ones1, zeros1)
    dego3 = dego.reshape(NC, 1, N8)
    degi3 = degi.reshape(NC, 1, N8)
    gcnt3 = gcnt.reshape(NC, 1, G8)
    gid3 = gid_tc.reshape(NC, 1, N8)
    xs = _tc_norm(x_all, dego3)
    m1 = _sc_scatter(xs.reshape(NC * N8, D), src_g, dst_l, zeros128)
    hs = _tc_mid(m1, degi3, dego3, Wc1, b_(bc1))
    m2 = _sc_scatter(hs.reshape(NC * N8, D), src_g, dst_l, zeros128)
    pools = _tc_pool(m2, degi3, Wc2, b_(bc2), gid3)
    out = _tc_mlp(pools, gcnt3, W1, b_(B1), W2, b_(B2), W3, b_(B3), W4p, B4p)
    return out[:, :C]


# speedup vs baseline: 3.7895x; 1.1762x over previous
import functools

import jax
import jax.numpy as jnp
from jax import lax
from jax.experimental import pallas as pl
from jax.experimental.pallas import tpu as pltpu
from jax.experimental.pallas import tpu_sc as plsc

N = 10000
E = 320000
D = 128
G = 256
C = 2

NC = 2
NS = 16
CH = 128
KE = 160
SLAB = 32
NSLAB = KE // SLAB
EPAD = NS * KE * CH
N8 = 10240
NPW = N8 // NS
KG = 5
G8 = 512
GPW = G8 // NS
F32 = jnp.float32
I32 = jnp.int32

_BR = 1280


def _sc_degrees_body(src_hbm, dst_hbm, gid_hbm, ones_hbm, zeros_hbm,
                     dego_hbm, degi_hbm, gcnt_hbm,
                     src_v, dst_v, gid_v, ones_v, zeros_v,
                     dego_s, degi_s, gcnt_s):
    c = lax.axis_index("c")
    s = lax.axis_index("s")
    pltpu.sync_copy(src_hbm.at[c, s], src_v)
    pltpu.sync_copy(dst_hbm.at[c, s], dst_v)
    pltpu.sync_copy(gid_hbm.at[c, s], gid_v)
    pltpu.sync_copy(ones_hbm, ones_v)
    pltpu.sync_copy(zeros_hbm, zeros_v)
    base = s * NPW
    sl = pl.ds(base, NPW)
    pltpu.sync_copy(zeros_v, dego_s.at[sl])
    pltpu.sync_copy(zeros_v, degi_s.at[sl])
    pltpu.sync_copy(zeros_v.at[pl.ds(0, GPW)], gcnt_s.at[pl.ds(s * GPW, GPW)])
    plsc.subcore_barrier()

    def edge_body(j, carry):
        pltpu.sync_copy(ones_v, dego_s.at[src_v.at[j]], add=True)
        pltpu.sync_copy(ones_v, degi_s.at[dst_v.at[j]], add=True)
        return carry

    lax.fori_loop(0, KE, edge_body, 0)

    def gid_body(j, carry):
        pltpu.sync_copy(ones_v, gcnt_s.at[gid_v.at[j]], add=True)
        return carry

    lax.fori_loop(0, KG, gid_body, 0)
    plsc.subcore_barrier()
    pltpu.sync_copy(dego_s.at[sl], zeros_v)
    pltpu.sync_copy(zeros_v, dego_hbm.at[c, sl])
    pltpu.sync_copy(degi_s.at[sl], zeros_v)
    pltpu.sync_copy(zeros_v, degi_hbm.at[c, sl])

    @pl.when(s < G8 // CH)
    def _():
        gsl_in = pl.ds(s * CH, CH)
        pltpu.sync_copy(gcnt_s.at[gsl_in], zeros_v.at[pl.ds(0, CH)])
        pltpu.sync_copy(zeros_v.at[pl.ds(0, CH)], gcnt_hbm.at[c, gsl_in])


def _sc_scatter_body(table_hbm, src_hbm, dst_hbm, zeros_hbm, out_hbm,
                     src_v, dst_v, gbufa, gbufb, acc_s, sema, semb):
    c = lax.axis_index("c")
    s = lax.axis_index("s")
    pltpu.sync_copy(zeros_hbm, gbufa)
    base = s * NPW
    for k in range(NPW // CH):
        pltpu.sync_copy(gbufa, acc_s.at[pl.ds(base + k * CH, CH)])
    plsc.subcore_barrier()

    for t in range(NSLAB):
        pltpu.sync_copy(src_hbm.at[c, s, pl.ds(t * SLAB, SLAB)], src_v)
        pltpu.sync_copy(dst_hbm.at[c, s, pl.ds(t * SLAB, SLAB)], dst_v)
        pltpu.async_copy(table_hbm.at[src_v.at[0]], gbufa, sema)

        def body2(jj, carry):
            j0 = 2 * jj
            pltpu.async_copy(table_hbm.at[src_v.at[j0 + 1]], gbufb, semb)
            pltpu.make_async_copy(table_hbm.at[src_v.at[j0]],
                                  gbufa, sema).wait()
            pltpu.sync_copy(gbufa, acc_s.at[dst_v.at[j0]], add=True)

            @pl.when(jj < SLAB // 2 - 1)
            def _():
                pltpu.async_copy(table_hbm.at[src_v.at[j0 + 2]], gbufa, sema)

            pltpu.make_async_copy(table_hbm.at[src_v.at[j0 + 1]],
                                  gbufb, semb).wait()
            pltpu.sync_copy(gbufb, acc_s.at[dst_v.at[j0 + 1]], add=True)
            return carry

        lax.fori_loop(0, SLAB // 2, body2, 0)
    plsc.subcore_barrier()
    for k in range(NPW // CH):
        sl = pl.ds(base + k * CH, CH)
        pltpu.sync_copy(acc_s.at[sl], gbufa)
        pltpu.sync_copy(gbufa, out_hbm.at[c, sl])


@functools.lru_cache(maxsize=None)
def _sc_kernels():
    mesh = plsc.VectorSubcoreMesh(core_axis_name="c", subcore_axis_name="s")
    degrees = pl.kernel(
        _sc_degrees_body,
        out_type=(jax.ShapeDtypeStruct((NC, N8), F32),
                  jax.ShapeDtypeStruct((NC, N8), F32),
                  jax.ShapeDtypeStruct((NC, G8), F32)),
        mesh=mesh,
        scratch_types=[
            pltpu.VMEM((KE, CH), I32),
            pltpu.VMEM((KE, CH), I32),
            pltpu.VMEM((KG, CH), I32),
            pltpu.VMEM((CH,), F32),
            pltpu.VMEM((NPW,), F32),
            pltpu.VMEM_SHARED((N8,), F32),
            pltpu.VMEM_SHARED((N8,), F32),
            pltpu.VMEM_SHARED((G8,), F32),
        ],
    )
    scatter = pl.kernel(
        _sc_scatter_body,
        out_type=jax.ShapeDtypeStruct((NC, N8, D), F32),
        mesh=mesh,
        scratch_types=[
            pltpu.VMEM((SLAB, CH), I32),
            pltpu.VMEM((SLAB, CH), I32),
            pltpu.VMEM((CH, D), F32),
            pltpu.VMEM((CH, D), F32),
            pltpu.VMEM_SHARED((N8, D), F32),
            pltpu.SemaphoreType.DMA,
            pltpu.SemaphoreType.DMA,
        ],
    )
    return degrees, scatter


def _col(v):
    return jnp.reshape(v, (v.shape[-1], 1))


def _tc_norm_body(x_ref, rso_ref, xs_ref):
    xs_ref[0] = x_ref[0] * _col(rso_ref[...])


_tc_norm = pl.pallas_call(
    _tc_norm_body,
    grid=(NC, N8 // _BR),
    in_specs=[pl.BlockSpec((1, _BR, D), lambda c, r: (c, r, 0)),
              pl.BlockSpec((1, 1, _BR), lambda c, r: (c, 0, r))],
    out_specs=pl.BlockSpec((1, _BR, D), lambda c, r: (c, r, 0)),
    out_shape=jax.ShapeDtypeStruct((NC, N8, D), F32),
)


def _tc_mid_body(m_ref, rsi_ref, rso_ref, w_ref, b_ref, out_ref):
    rsdi = _col(rsi_ref[...])
    rsdo = _col(rso_ref[...])
    h = jnp.dot(m_ref[0] * rsdi, w_ref[...], preferred_element_type=F32)
    h = jnp.maximum(h + b_ref[...], 0.0)
    out_ref[0] = h * rsdo


_tc_mid = pl.pallas_call(
    _tc_mid_body,
    grid=(NC, N8 // _BR),
    in_specs=[pl.BlockSpec((1, _BR, D), lambda c, r: (c, r, 0)),
              pl.BlockSpec((1, 1, _BR), lambda c, r: (c, 0, r)),
              pl.BlockSpec((1, 1, _BR), lambda c, r: (c, 0, r)),
              pl.BlockSpec((D, D), lambda c, r: (0, 0)),
              pl.BlockSpec((1, D), lambda c, r: (0, 0))],
    out_specs=pl.BlockSpec((1, _BR, D), lambda c, r: (c, r, 0)),
    out_shape=jax.ShapeDtypeStruct((NC, N8, D), F32),
)


def _tc_pool_body(m_ref, rsi_ref, w_ref, b_ref, gid_ref, out_ref):
    r = pl.program_id(1)
    rsdi = _col(rsi_ref[...])
    h = jnp.dot(m_ref[0] * rsdi, w_ref[...], preferred_element_type=F32)
    h = jnp.maximum(h + b_ref[...], 0.0)
    gid = jnp.reshape(gid_ref[...], (1, _BR))
    oh = (lax.broadcasted_iota(I32, (G, _BR), 0) == gid).astype(F32)
    contrib = jnp.dot(oh, h, preferred_element_type=F32,
                      precision=lax.Precision.HIGHEST)

    @pl.when(r == 0)
    def _():
        out_ref[0] = contrib

    @pl.when(r != 0)
    def _():
        out_ref[0] += contrib


_tc_pool = pl.pallas_call(
    _tc_pool_body,
    grid=(NC, N8 // _BR),
    in_specs=[pl.BlockSpec((1, _BR, D), lambda c, r: (c, r, 0)),
              pl.BlockSpec((1, 1, _BR), lambda c, r: (c, 0, r)),
              pl.BlockSpec((D, D), lambda c, r: (0, 0)),
              pl.BlockSpec((1, D), lambda c, r: (0, 0)),
              pl.BlockSpec((1, 1, _BR), lambda c, r: (c, 0, r))],
    out_specs=pl.BlockSpec((1, G, D), lambda c, r: (c, 0, 0)),
    out_shape=jax.ShapeDtypeStruct((NC, G, D), F32),
    compiler_params=pltpu.CompilerParams(
        dimension_semantics=("arbitrary", "arbitrary")),
)


def _tc_mlp_body(p_ref, gcnt_ref, w1, b1, w2, b2, w3, b3, w4, b4, out_ref):
    cnt = gcnt_ref[...]
    c1 = _col(jnp.maximum(cnt[0, :, :G], 1.0))
    c2 = _col(jnp.maximum(cnt[1, :, :G], 1.0))
    hg = jnp.concatenate([p_ref[0] / c1, p_ref[1] / c2], axis=1)
    o = jnp.maximum(jnp.dot(hg, w1[...], preferred_element_type=F32) + b1[...], 0.0)
    o = jnp.maximum(jnp.dot(o, w2[...], preferred_element_type=F32) + b2[...], 0.0)
    o = jnp.maximum(jnp.dot(o, w3[...], preferred_element_type=F32) + b3[...], 0.0)
    out_ref[...] = jnp.dot(o, w4[...], preferred_element_type=F32) + b4[...]


_tc_mlp = pl.pallas_call(
    _tc_mlp_body,
    out_shape=jax.ShapeDtypeStruct((G, D), F32),
)


def kernel(x1, x2, edge_index1, edge_index2, graph_id1, graph_id2,
           Wc1, bc1, Wc2, bc2, W1, B1, W2, B2, W3, B3, W4, B4):
    def prep(ei):
        srcp = jnp.concatenate([ei[0], jnp.full((EPAD - E,), N, I32)])
        dstp = jnp.concatenate([ei[1], jnp.full((EPAD - E,), N, I32)])
        return srcp.reshape(NS, KE, CH), dstp.reshape(NS, KE, CH)

    s1, d1 = prep(edge_index1)
    s2, d2 = prep(edge_index2)
    src_l = jnp.stack([s1, s2])
    dst_l = jnp.stack([d1, d2])
    src_g = src_l + (jnp.arange(NC, dtype=I32) * N8)[:, None, None, None]
    gidp = jnp.stack([
        jnp.concatenate([graph_id1, jnp.full((N8 - N,), G, I32)]),
        jnp.concatenate([graph_id2, jnp.full((N8 - N,), G, I32)])])
    gid_sc = gidp.reshape(NC, NS, KG, CH)
    gid_tc = gidp
    x_all = jnp.zeros((NC, N8, D), F32).at[:, :N, :].set(jnp.stack([x1, x2]))
    ones1 = jnp.ones((CH,), F32)
    zeros1 = jnp.zeros((NPW,), F32)
    zeros128 = jnp.zeros((CH, D), F32)
    W4p = jnp.zeros((D, D), F32).at[:, :C].set(W4)
    B4p = jnp.zeros((1, D), F32).at[0, :C].set(B4)

    def b_(v):
        return v.reshape(1, -1)

    _sc_degrees, _sc_scatter = _sc_kernels()
    dego, degi, gcnt = _sc_degrees(src_l, dst_l, gid_sc, ones1, zeros1)
    rso3 = (jnp.clip(dego, 1.0, None) ** -0.5).reshape(NC, 1, N8)
    rsi3 = (jnp.clip(degi, 1.0, None) ** -0.5).reshape(NC, 1, N8)
    gcnt3 = gcnt.reshape(NC, 1, G8)
    gid3 = gid_tc.reshape(NC, 1, N8)
    xs = _tc_norm(x_all, rso3)
    m1 = _sc_scatter(xs.reshape(NC * N8, D), src_g, dst_l, zeros128)
    hs = _tc_mid(m1, rsi3, rso3, Wc1, b_(bc1))
    m2 = _sc_scatter(hs.reshape(NC * N8, D), src_g, dst_l, zeros128)
    pools = _tc_pool(m2, rsi3, Wc2, b_(bc2), gid3)
    out = _tc_mlp(pools, gcnt3, W1, b_(B1), W2, b_(B2), W3, b_(B3), W4p, B4p)
    return out[:, :C]
